# baseline (device time: 7618 ns/iter reference)
import functools

import jax
import jax.numpy as jnp
from jax import lax
from jax.experimental import pallas as pl
from jax.experimental.pallas import tpu as pltpu

MESH_X = 2
MESH_Y = 2


def kernel(x):
    m, n = x.shape
    gm, gn = MESH_X * m, MESH_Y * n

    def body(
        x_ref, out_ref, row_send, row_recv, col_send, col_recv,
        send_sems, recv_sems,
    ):
        my_x = lax.axis_index("x")
        my_y = lax.axis_index("y")
        nbr_x = MESH_X - 1 - my_x
        nbr_y = MESH_Y - 1 - my_y

        xv = x_ref[:, :]

        row_send[:, :] = jnp.where(my_x == 0, xv[m - 1 : m, :], xv[0:1, :])
        colv = jnp.where(my_y == 0, xv[:, n - 1 : n], xv[:, 0:1])
        col_send[:, :] = jnp.transpose(colv, (1, 0))

        barrier_sem = pltpu.get_barrier_semaphore()
        pl.semaphore_signal(
            barrier_sem, inc=1,
            device_id=(nbr_x, my_y), device_id_type=pl.DeviceIdType.MESH,
        )
        pl.semaphore_signal(
            barrier_sem, inc=1,
            device_id=(my_x, nbr_y), device_id_type=pl.DeviceIdType.MESH,
        )
        pl.semaphore_wait(barrier_sem, 2)

        rdma_row = pltpu.make_async_remote_copy(
            src_ref=row_send,
            dst_ref=row_recv,
            send_sem=send_sems.at[0],
            recv_sem=recv_sems.at[0],
            device_id=(nbr_x, my_y),
            device_id_type=pl.DeviceIdType.MESH,
        )
        rdma_col = pltpu.make_async_remote_copy(
            src_ref=col_send,
            dst_ref=col_recv,
            send_sem=send_sems.at[1],
            recv_sem=recv_sems.at[1],
            device_id=(my_x, nbr_y),
            device_id_type=pl.DeviceIdType.MESH,
        )
        rdma_row.start()
        rdma_col.start()

        up0 = jnp.concatenate([xv[0:1, :], xv[:-1, :]], axis=0)
        down0 = jnp.concatenate([xv[1:, :], xv[m - 1 : m, :]], axis=0)
        left0 = jnp.concatenate([xv[:, 0:1], xv[:, :-1]], axis=1)
        right0 = jnp.concatenate([xv[:, 1:], xv[:, n - 1 : n]], axis=1)
        sten0 = 0.5 * xv + 0.125 * (up0 + down0 + left0 + right0)

        gr = lax.broadcasted_iota(jnp.int32, (m, n), 0) + my_x * m
        gc = lax.broadcasted_iota(jnp.int32, (m, n), 1) + my_y * n
        boundary = (gr == 0) | (gr == gm - 1) | (gc == 0) | (gc == gn - 1)
        out_ref[:, :] = jnp.where(boundary, xv, sten0)

        rdma_row.wait()
        rdma_col.wait()

        rbuf = row_recv[:, :]
        cbuf = col_recv[:, :]

        def fix_row(r0, up, down):
            row_x = xv[r0 : r0 + 1, :]
            edge_l = jnp.where(my_y == 1, cbuf[0:1, r0 : r0 + 1], row_x[:, 0:1])
            edge_r = jnp.where(
                my_y == 0, cbuf[0:1, r0 : r0 + 1], row_x[:, n - 1 : n]
            )
            left = jnp.concatenate([edge_l, row_x[:, :-1]], axis=1)
            right = jnp.concatenate([row_x[:, 1:], edge_r], axis=1)
            sten = 0.5 * row_x + 0.125 * (up + down + left + right)
            gc_r = lax.broadcasted_iota(jnp.int32, (1, n), 1) + my_y * n
            mask = (gc_r == 0) | (gc_r == gn - 1)
            out_ref[r0 : r0 + 1, :] = jnp.where(mask, row_x, sten)

        @pl.when(my_x == 0)
        def _():
            fix_row(m - 1, xv[m - 2 : m - 1, :], rbuf)

        @pl.when(my_x == 1)
        def _():
            fix_row(0, rbuf, xv[1:2, :])

        cbuf_col = jnp.transpose(cbuf, (1, 0))

        def fix_col(c0, left, right):
            col_x = xv[:, c0 : c0 + 1]
            edge_u = jnp.where(my_x == 1, rbuf[0:1, c0 : c0 + 1], col_x[0:1, :])
            edge_d = jnp.where(
                my_x == 0, rbuf[0:1, c0 : c0 + 1], col_x[m - 1 : m, :]
            )
            up = jnp.concatenate([edge_u, col_x[:-1, :]], axis=0)
            down = jnp.concatenate([col_x[1:, :], edge_d], axis=0)
            sten = 0.5 * col_x + 0.125 * (up + down + left + right)
            gr_c = lax.broadcasted_iota(jnp.int32, (m, 1), 0) + my_x * m
            mask = (gr_c == 0) | (gr_c == gm - 1)
            out_ref[:, c0 : c0 + 1] = jnp.where(mask, col_x, sten)

        @pl.when(my_y == 0)
        def _():
            fix_col(n - 1, xv[:, n - 2 : n - 1], cbuf_col)

        @pl.when(my_y == 1)
        def _():
            fix_col(0, cbuf_col, xv[:, 1:2])

        @functools.partial(
            pl.run_scoped, second_barrier=pltpu.SemaphoreType.REGULAR
        )
        def _(second_barrier):
            pl.semaphore_signal(
                second_barrier, inc=1,
                device_id=(nbr_x, my_y), device_id_type=pl.DeviceIdType.MESH,
            )
            pl.semaphore_signal(
                second_barrier, inc=1,
                device_id=(my_x, nbr_y), device_id_type=pl.DeviceIdType.MESH,
            )
            pl.semaphore_wait(second_barrier, 2)

    return pl.pallas_call(
        body,
        out_shape=jax.ShapeDtypeStruct((m, n), x.dtype),
        in_specs=[pl.BlockSpec(memory_space=pltpu.VMEM)],
        out_specs=pl.BlockSpec(memory_space=pltpu.VMEM),
        scratch_shapes=[
            pltpu.VMEM((1, n), x.dtype),
            pltpu.VMEM((1, n), x.dtype),
            pltpu.VMEM((1, m), x.dtype),
            pltpu.VMEM((1, m), x.dtype),
            pltpu.SemaphoreType.DMA((2,)),
            pltpu.SemaphoreType.DMA((2,)),
        ],
        compiler_params=pltpu.CompilerParams(collective_id=0),
    )(x)
